# baseline (speedup 1.0000x reference)
import functools

import jax
import jax.numpy as jnp
from jax import lax
from jax.experimental import pallas as pl
from jax.experimental.pallas import tpu as pltpu
from jax.experimental.pallas import tpu_sc as plsc

_VOCAB = 1000000
_DIM = 64
_UNITS = 256
_BATCH = 1024
_SEQ = 50

_NC = 2
_NS = 16
_NW = _NC * _NS
_CHUNK = 80
_NCHUNK = (_BATCH * _SEQ) // (_NW * _CHUNK)


def _sc_gather(table2, idx3):
    mesh = plsc.VectorSubcoreMesh(core_axis_name="c", subcore_axis_name="s")

    @functools.partial(
        pl.kernel,
        mesh=mesh,
        compiler_params=pltpu.CompilerParams(use_tc_tiling_on_sc=False),
        out_type=jax.ShapeDtypeStruct((_NW, _NCHUNK, _CHUNK, 128), jnp.float32),
        scratch_types=[
            pltpu.VMEM((_NCHUNK, _CHUNK), jnp.int32),
            pltpu.VMEM((_NCHUNK // 4, _CHUNK, 128), jnp.float32),
            pltpu.VMEM((_NCHUNK // 4, _CHUNK, 128), jnp.float32),
            pltpu.SemaphoreType.DMA,
            pltpu.SemaphoreType.DMA,
        ],
    )
    def gather_kernel(table_hbm, idx_hbm, out_hbm, idx_v, rows_a, rows_b, sem_g, sem_s):
        wid = lax.axis_index("s") * _NC + lax.axis_index("c")
        pltpu.sync_copy(idx_hbm.at[wid], idx_v)
        bufs = (rows_a, rows_b)
        n_pass = 4
        k = _NCHUNK // n_pass
        stores = [None, None]
        for p in range(n_pass):
            buf = bufs[p % 2]
            if stores[p % 2] is not None:
                stores[p % 2].wait()
            gathers = [
                pltpu.async_copy(
                    table_hbm.at[idx_v.at[p * k + j]], buf.at[j], sem_g
                )
                for j in range(k)
            ]
            for g in gathers:
                g.wait()
            stores[p % 2] = pltpu.async_copy(
                buf, out_hbm.at[wid, pl.ds(p * k, k)], sem_s
            )
        for st in stores:
            if st is not None:
                st.wait()

    return gather_kernel(table2, idx3)


_TCOLS = 16384


_TGRID = (_VOCAB + _TCOLS - 1) // _TCOLS
_NPAIR = _TGRID * (_TCOLS // 2)


def _pack_step(in_ref, out_ref):
    t = in_ref[...]
    lo = lax.bitcast_convert_type(t[:32].astype(jnp.bfloat16), jnp.uint16)
    hi = lax.bitcast_convert_type(t[32:].astype(jnp.bfloat16), jnp.uint16)
    w = lax.bitcast_convert_type(
        lo.astype(jnp.uint32) | (hi.astype(jnp.uint32) << 16), jnp.float32
    )
    q = _TCOLS // 4
    out_ref[...] = jnp.concatenate(
        [w[:, :q].T, w[:, q:2 * q].T, w[:, 2 * q:3 * q].T, w[:, 3 * q:].T],
        axis=1,
    )


def _tc_pack_quads(tabT):
    return pl.pallas_call(
        _pack_step,
        grid=(_TGRID,),
        in_specs=[pl.BlockSpec((_DIM, _TCOLS), lambda i: (0, i))],
        out_specs=pl.BlockSpec((_TCOLS // 4, 128), lambda i: (i, 0)),
        out_shape=jax.ShapeDtypeStruct((_TGRID * (_TCOLS // 4), 128), jnp.float32),
    )(tabT)


def _gru_step(pair_ref, par_ref, w_ref, u_ref, b_ref, h_ref):
    t = pl.program_id(0)

    @pl.when(t == 0)
    def _():
        h_ref[...] = jnp.zeros_like(h_ref)

    h = h_ref[...]
    quad = pair_ref[0]
    par = par_ref[0]
    halfsel = jnp.where(par >= 2, quad[:, 64:], quad[:, :64])
    packed = jnp.where(par % 2 == 1, halfsel[:, 32:], halfsel[:, :32])
    u = lax.bitcast_convert_type(packed, jnp.uint32)
    flo = lax.bitcast_convert_type((u & 0xFFFF).astype(jnp.uint16),
                                   jnp.bfloat16)
    fhi = lax.bitcast_convert_type((u >> 16).astype(jnp.uint16),
                                   jnp.bfloat16)
    xt = jnp.concatenate([flo, fhi], axis=1)
    bb = b_ref[...]
    xp = jnp.dot(xt, w_ref[...], preferred_element_type=jnp.float32) + bb[0:1]
    rp = jnp.dot(h, u_ref[...], preferred_element_type=jnp.float32) + bb[1:2]
    xz = xp[:, :_UNITS]
    xr = xp[:, _UNITS:2 * _UNITS]
    xh = xp[:, 2 * _UNITS:]
    rz = rp[:, :_UNITS]
    rr = rp[:, _UNITS:2 * _UNITS]
    rh = rp[:, 2 * _UNITS:]
    z = jax.nn.sigmoid(xz + rz)
    r = jax.nn.sigmoid(xr + rr)
    hh = jnp.tanh(xh + r * rh)
    h_ref[...] = z * h + (1.0 - z) * hh


def _tc_gru(pairs, parity, W, U, b):
    return pl.pallas_call(
        _gru_step,
        grid=(_SEQ,),
        in_specs=[
            pl.BlockSpec((1, _BATCH, 128), lambda t: (t, 0, 0)),
            pl.BlockSpec((1, _BATCH, 1), lambda t: (t, 0, 0)),
            pl.BlockSpec((_DIM, 3 * _UNITS), lambda t: (0, 0)),
            pl.BlockSpec((_UNITS, 3 * _UNITS), lambda t: (0, 0)),
            pl.BlockSpec((2, 3 * _UNITS), lambda t: (0, 0)),
        ],
        out_specs=pl.BlockSpec((_BATCH, _UNITS), lambda t: (0, 0)),
        out_shape=jax.ShapeDtypeStruct((_BATCH, _UNITS), jnp.float32),
    )(pairs, parity, W, U, b)


def kernel(x, emb_table, W, U, b):
    table2 = _tc_pack_quads(jnp.transpose(emb_table))
    idx = jnp.transpose(x).reshape(_NW, _NCHUNK, _CHUNK)
    q = _TCOLS // 4
    m = idx % _TCOLS
    row_id = (idx // _TCOLS) * q + m % q
    quarter = m // q
    rows = _sc_gather(table2, row_id)
    quads = rows.reshape(_SEQ, _BATCH, 128)
    parity = quarter.reshape(_SEQ, _BATCH, 1)
    return _tc_gru(quads, parity, W.astype(jnp.bfloat16), U, b)


# speedup vs baseline: 1.0839x; 1.0839x over previous
import functools

import jax
import jax.numpy as jnp
from jax import lax
from jax.experimental import pallas as pl
from jax.experimental.pallas import tpu as pltpu
from jax.experimental.pallas import tpu_sc as plsc

_VOCAB = 1000000
_DIM = 64
_UNITS = 256
_BATCH = 1024
_SEQ = 50

_NC = 2
_NS = 16
_NW = _NC * _NS
_CHUNK = 80
_NCHUNK = (_BATCH * _SEQ) // (_NW * _CHUNK)


def _sc_gather(table2, idx3):
    mesh = plsc.VectorSubcoreMesh(core_axis_name="c", subcore_axis_name="s")

    @functools.partial(
        pl.kernel,
        mesh=mesh,
        compiler_params=pltpu.CompilerParams(use_tc_tiling_on_sc=False),
        out_type=jax.ShapeDtypeStruct((_NW, _NCHUNK, _CHUNK, 128), jnp.float32),
        scratch_types=[
            pltpu.VMEM((_NCHUNK, _CHUNK), jnp.int32),
            pltpu.VMEM((_NCHUNK // 4, _CHUNK, 128), jnp.float32),
            pltpu.VMEM((_NCHUNK // 4, _CHUNK, 128), jnp.float32),
            pltpu.SemaphoreType.DMA,
            pltpu.SemaphoreType.DMA,
        ],
    )
    def gather_kernel(table_hbm, idx_hbm, out_hbm, idx_v, rows_a, rows_b, sem_g, sem_s):
        wid = lax.axis_index("s") * _NC + lax.axis_index("c")
        pltpu.sync_copy(idx_hbm.at[wid], idx_v)
        bufs = (rows_a, rows_b)
        n_pass = 4
        k = _NCHUNK // n_pass
        stores = [None, None]
        for p in range(n_pass):
            buf = bufs[p % 2]
            if stores[p % 2] is not None:
                stores[p % 2].wait()
            gathers = [
                pltpu.async_copy(
                    table_hbm.at[idx_v.at[p * k + j]], buf.at[j], sem_g
                )
                for j in range(k)
            ]
            for g in gathers:
                g.wait()
            stores[p % 2] = pltpu.async_copy(
                buf, out_hbm.at[wid, pl.ds(p * k, k)], sem_s
            )
        for st in stores:
            if st is not None:
                st.wait()

    return gather_kernel(table2, idx3)


_TCOLS = 16384


_TGRID = (_VOCAB + _TCOLS - 1) // _TCOLS
_NPAIR = _TGRID * (_TCOLS // 2)


def _pack_step(in_ref, out_ref):
    t = in_ref[...]
    h = _TCOLS // 2
    out_ref[:, :_DIM] = t[:, :h].T
    out_ref[:, _DIM:] = t[:, h:].T


def _tc_pack_pairs(tabT):
    return pl.pallas_call(
        _pack_step,
        grid=(_TGRID,),
        in_specs=[pl.BlockSpec((_DIM, _TCOLS), lambda i: (0, i))],
        out_specs=pl.BlockSpec((_TCOLS // 2, 128), lambda i: (i, 0)),
        out_shape=jax.ShapeDtypeStruct((_NPAIR, 128), jnp.float32),
    )(tabT)


def _gru_step(pair_ref, par_ref, w_ref, u_ref, b_ref, h_ref):
    t = pl.program_id(0)

    @pl.when(t == 0)
    def _():
        h_ref[...] = jnp.zeros_like(h_ref)

    h = h_ref[...]
    pair = pair_ref[0]
    parity = par_ref[0]
    xt = jnp.where(parity > 0, pair[:, _DIM:], pair[:, :_DIM])
    bb = b_ref[...]
    xp = jnp.dot(xt.astype(jnp.bfloat16), w_ref[...],
                 preferred_element_type=jnp.float32) + bb[0:1]
    rp = jnp.dot(h.astype(jnp.bfloat16), u_ref[...],
                 preferred_element_type=jnp.float32) + bb[1:2]
    xz = xp[:, :_UNITS]
    xr = xp[:, _UNITS:2 * _UNITS]
    xh = xp[:, 2 * _UNITS:]
    rz = rp[:, :_UNITS]
    rr = rp[:, _UNITS:2 * _UNITS]
    rh = rp[:, 2 * _UNITS:]
    z = jax.nn.sigmoid(xz + rz)
    r = jax.nn.sigmoid(xr + rr)
    hh = jnp.tanh(xh + r * rh)
    h_ref[...] = z * h + (1.0 - z) * hh


def _tc_gru(pairs, parity, W, U, b):
    return pl.pallas_call(
        _gru_step,
        grid=(_SEQ,),
        in_specs=[
            pl.BlockSpec((1, _BATCH, 128), lambda t: (t, 0, 0)),
            pl.BlockSpec((1, _BATCH, 1), lambda t: (t, 0, 0)),
            pl.BlockSpec((_DIM, 3 * _UNITS), lambda t: (0, 0)),
            pl.BlockSpec((_UNITS, 3 * _UNITS), lambda t: (0, 0)),
            pl.BlockSpec((2, 3 * _UNITS), lambda t: (0, 0)),
        ],
        out_specs=pl.BlockSpec((_BATCH, _UNITS), lambda t: (0, 0)),
        out_shape=jax.ShapeDtypeStruct((_BATCH, _UNITS), jnp.float32),
    )(pairs, parity, W, U, b)


def kernel(x, emb_table, W, U, b):
    table2 = _tc_pack_pairs(jnp.transpose(emb_table))
    idx = jnp.transpose(x).reshape(_NW, _NCHUNK, _CHUNK)
    h = _TCOLS // 2
    pair = (idx // _TCOLS) * h + (idx % _TCOLS) % h
    half = (idx % _TCOLS) // h
    rows = _sc_gather(table2, pair)
    pairs = rows.reshape(_SEQ, _BATCH, 2 * _DIM)
    parity = half.reshape(_SEQ, _BATCH, 1)
    return _tc_gru(pairs, parity, W.astype(jnp.bfloat16),
                   U.astype(jnp.bfloat16), b)


# speedup vs baseline: 1.0939x; 1.0092x over previous
import functools

import jax
import jax.numpy as jnp
from jax import lax
from jax.experimental import pallas as pl
from jax.experimental.pallas import tpu as pltpu
from jax.experimental.pallas import tpu_sc as plsc

_VOCAB = 1000000
_DIM = 64
_UNITS = 256
_BATCH = 1024
_SEQ = 50

_NC = 2
_NS = 16
_NW = _NC * _NS
_CHUNK = 80
_NCHUNK = (_BATCH * _SEQ) // (_NW * _CHUNK)


def _sc_gather(table2, idx3):
    mesh = plsc.VectorSubcoreMesh(core_axis_name="c", subcore_axis_name="s")

    @functools.partial(
        pl.kernel,
        mesh=mesh,
        compiler_params=pltpu.CompilerParams(use_tc_tiling_on_sc=False),
        out_type=jax.ShapeDtypeStruct((_NW, _NCHUNK, _CHUNK, 128), jnp.float32),
        scratch_types=[
            pltpu.VMEM((_NCHUNK, _CHUNK), jnp.int32),
            pltpu.VMEM((_NCHUNK // 4, _CHUNK, 128), jnp.float32),
            pltpu.VMEM((_NCHUNK // 4, _CHUNK, 128), jnp.float32),
            pltpu.SemaphoreType.DMA,
            pltpu.SemaphoreType.DMA,
        ],
    )
    def gather_kernel(table_hbm, idx_hbm, out_hbm, idx_v, rows_a, rows_b, sem_g, sem_s):
        wid = lax.axis_index("s") * _NC + lax.axis_index("c")
        pltpu.sync_copy(idx_hbm.at[wid], idx_v)
        bufs = (rows_a, rows_b)
        n_pass = 4
        k = _NCHUNK // n_pass
        stores = [None, None]
        for p in range(n_pass):
            buf = bufs[p % 2]
            if stores[p % 2] is not None:
                stores[p % 2].wait()
            gathers = [
                pltpu.async_copy(
                    table_hbm.at[idx_v.at[p * k + j]], buf.at[j], sem_g
                )
                for j in range(k)
            ]
            for g in gathers:
                g.wait()
            stores[p % 2] = pltpu.async_copy(
                buf, out_hbm.at[wid, pl.ds(p * k, k)], sem_s
            )
        for st in stores:
            if st is not None:
                st.wait()

    return gather_kernel(table2, idx3)


_TCOLS = 16384


_TGRID = (_VOCAB + _TCOLS - 1) // _TCOLS
_NPAIR = _TGRID * (_TCOLS // 2)


def _pack_step(in_ref, out_ref):
    t = in_ref[...]
    h = _TCOLS // 2
    out_ref[:, :_DIM] = t[:, :h].T
    out_ref[:, _DIM:] = t[:, h:].T


def _tc_pack_pairs(tabT):
    return pl.pallas_call(
        _pack_step,
        grid=(_TGRID,),
        in_specs=[pl.BlockSpec((_DIM, _TCOLS), lambda i: (0, i))],
        out_specs=pl.BlockSpec((_TCOLS // 2, 128), lambda i: (i, 0)),
        out_shape=jax.ShapeDtypeStruct((_NPAIR, 128), jnp.float32),
    )(tabT)


def _gru_step(pair_ref, par_ref, w_ref, u_ref, b_ref, h_ref):
    t = pl.program_id(0)

    @pl.when(t == 0)
    def _():
        h_ref[...] = jnp.zeros_like(h_ref)

    h = h_ref[...]
    pair = pair_ref[0]
    parity = par_ref[0]
    xt = jnp.where(parity > 0, pair[:, _DIM:], pair[:, :_DIM])
    bb = b_ref[...]
    xp = jnp.dot(xt, w_ref[...], preferred_element_type=jnp.float32) + bb[0:1]
    rp = jnp.dot(h, u_ref[...], preferred_element_type=jnp.float32) + bb[1:2]
    xz = xp[:, :_UNITS]
    xr = xp[:, _UNITS:2 * _UNITS]
    xh = xp[:, 2 * _UNITS:]
    rz = rp[:, :_UNITS]
    rr = rp[:, _UNITS:2 * _UNITS]
    rh = rp[:, 2 * _UNITS:]
    z = jax.nn.sigmoid(xz + rz)
    r = jax.nn.sigmoid(xr + rr)
    hh = jnp.tanh(xh + r * rh)
    h_ref[...] = z * h + (1.0 - z) * hh


def _tc_gru(pairs, parity, W, U, b):
    return pl.pallas_call(
        _gru_step,
        grid=(_SEQ,),
        in_specs=[
            pl.BlockSpec((1, _BATCH, 128), lambda t: (t, 0, 0)),
            pl.BlockSpec((1, _BATCH, 1), lambda t: (t, 0, 0)),
            pl.BlockSpec((_DIM, 3 * _UNITS), lambda t: (0, 0)),
            pl.BlockSpec((_UNITS, 3 * _UNITS), lambda t: (0, 0)),
            pl.BlockSpec((2, 3 * _UNITS), lambda t: (0, 0)),
        ],
        out_specs=pl.BlockSpec((_BATCH, _UNITS), lambda t: (0, 0)),
        out_shape=jax.ShapeDtypeStruct((_BATCH, _UNITS), jnp.float32),
    )(pairs, parity, W, U, b)


def kernel(x, emb_table, W, U, b):
    table2 = _tc_pack_pairs(jnp.transpose(emb_table))
    idx = jnp.transpose(x).reshape(_NW, _NCHUNK, _CHUNK)
    h = _TCOLS // 2
    pair = (idx // _TCOLS) * h + (idx % _TCOLS) % h
    half = (idx % _TCOLS) // h
    rows = _sc_gather(table2, pair)
    pairs = rows.reshape(_SEQ, _BATCH, 2 * _DIM)
    parity = half.reshape(_SEQ, _BATCH, 1)
    return _tc_gru(pairs, parity, W, U, b)
